# Initial kernel scaffold; baseline (speedup 1.0000x reference)
import functools

import jax
import jax.numpy as jnp
from jax import lax
from jax.experimental import pallas as pl
from jax.experimental.pallas import tpu as pltpu
from jax.experimental.pallas import tpu_sc as plsc

_EMB = 64
_CHUNK = 128


@functools.lru_cache(maxsize=None)
def _build(BL: int, L: int):
    info = plsc.get_sparse_core_info()
    NC, NS = info.num_cores, info.num_subcores
    NW = NC * NS
    assert BL % (NW * _CHUNK) == 0
    rows_w = BL // NW
    n_chunks = rows_w // _CHUNK
    mesh = plsc.VectorSubcoreMesh(core_axis_name="c", subcore_axis_name="s")

    @functools.partial(
        pl.kernel,
        mesh=mesh,
        out_type=jax.ShapeDtypeStruct((BL, _EMB), jnp.float32),
        scratch_types=[
            pltpu.VMEM((_CHUNK,), jnp.int32),
            pltpu.VMEM((L, _EMB), jnp.float32),
            pltpu.VMEM((_CHUNK, _EMB), jnp.float32),
            pltpu.SemaphoreType.DMA,
        ],
    )
    def k(x_hbm, emb_hbm, pos_hbm, out_hbm, idx_v, pos_v, buf_v, sem):
        cid = lax.axis_index("c")
        sid = lax.axis_index("s")
        wid = sid * NC + cid
        base = wid * rows_w
        pltpu.sync_copy(pos_hbm.at[pl.ds(0, L)], pos_v)

        def chunk_body(c, carry):
            rb = base + c * _CHUNK
            pltpu.sync_copy(x_hbm.at[pl.ds(rb, _CHUNK)], idx_v)
            pltpu.async_copy(emb_hbm.at[idx_v], buf_v, sem).wait()
            start = lax.rem(c * _CHUNK, L)

            def row_body(j, _):
                p = lax.rem(start + j, L)
                for e in range(_EMB // 16):
                    sl = pl.ds(e * 16, 16)
                    buf_v[j, sl] = buf_v[j, sl] + pos_v[p, sl]
                return 0

            lax.fori_loop(0, _CHUNK, row_body, 0)
            pltpu.sync_copy(buf_v, out_hbm.at[pl.ds(rb, _CHUNK)])
            return carry

        lax.fori_loop(0, n_chunks, chunk_body, 0)

    return k


def kernel(x, emb_table, pos_table):
    B, L = x.shape
    BL = B * L
    xf = x.reshape(BL).astype(jnp.int32)
    out = _build(BL, L)(xf, emb_table, pos_table)
    return out.reshape(B, L, _EMB)


# speedup vs baseline: 1.8305x; 1.8305x over previous
import functools

import jax
import jax.numpy as jnp
from jax import lax
from jax.experimental import pallas as pl
from jax.experimental.pallas import tpu as pltpu
from jax.experimental.pallas import tpu_sc as plsc

_EMB = 64
_CHUNK = 128


@functools.lru_cache(maxsize=None)
def _build(BL: int, L: int):
    info = plsc.get_sparse_core_info()
    NC, NS = info.num_cores, info.num_subcores
    NW = NC * NS
    assert BL % (NW * _CHUNK) == 0
    rows_w = BL // NW
    n_chunks = rows_w // _CHUNK
    mesh = plsc.VectorSubcoreMesh(core_axis_name="c", subcore_axis_name="s")

    @functools.partial(
        pl.kernel,
        mesh=mesh,
        compiler_params=pltpu.CompilerParams(use_tc_tiling_on_sc=False),
        out_type=jax.ShapeDtypeStruct((BL, _EMB), jnp.float32),
        scratch_types=[
            pltpu.VMEM((_CHUNK,), jnp.int32),
            pltpu.VMEM((L, _EMB), jnp.float32),
            pltpu.VMEM((_CHUNK, _EMB), jnp.float32),
            pltpu.SemaphoreType.DMA,
        ],
    )
    def k(x_hbm, emb_hbm, pos_hbm, out_hbm, idx_v, pos_v, buf_v, sem):
        cid = lax.axis_index("c")
        sid = lax.axis_index("s")
        wid = sid * NC + cid
        base = wid * rows_w
        pltpu.sync_copy(pos_hbm.at[pl.ds(0, L)], pos_v)

        def chunk_body(c, carry):
            rb = base + c * _CHUNK
            pltpu.sync_copy(x_hbm.at[pl.ds(rb, _CHUNK)], idx_v)
            pltpu.async_copy(emb_hbm.at[idx_v], buf_v, sem).wait()
            start = lax.rem(c * _CHUNK, L)

            def row_body(j, _):
                p = lax.rem(start + j, L)
                for e in range(_EMB // 16):
                    sl = pl.ds(e * 16, 16)
                    buf_v[j, sl] = buf_v[j, sl] + pos_v[p, sl]
                return 0

            lax.fori_loop(0, _CHUNK, row_body, 0)
            pltpu.sync_copy(buf_v, out_hbm.at[pl.ds(rb, _CHUNK)])
            return carry

        lax.fori_loop(0, n_chunks, chunk_body, 0)

    return k


def kernel(x, emb_table, pos_table):
    B, L = x.shape
    BL = B * L
    xf = x.reshape(BL).astype(jnp.int32)
    out = _build(BL, L)(xf, emb_table, pos_table)
    return out.reshape(B, L, _EMB)


# speedup vs baseline: 2.4242x; 1.3243x over previous
import functools

import jax
import jax.numpy as jnp
from jax import lax
from jax.experimental import pallas as pl
from jax.experimental.pallas import tpu as pltpu
from jax.experimental.pallas import tpu_sc as plsc

_EMB = 64
_CHUNK = 128
_NB = 4
_LOOK = 2


@functools.lru_cache(maxsize=None)
def _build(BL: int, L: int):
    info = plsc.get_sparse_core_info()
    NC, NS = info.num_cores, info.num_subcores
    NW = NC * NS
    assert BL % (NW * _CHUNK) == 0
    rows_w = BL // NW
    n_chunks = rows_w // _CHUNK
    assert n_chunks % _NB == 0 and n_chunks >= 2 * _NB
    mesh = plsc.VectorSubcoreMesh(core_axis_name="c", subcore_axis_name="s")

    @functools.partial(
        pl.kernel,
        mesh=mesh,
        compiler_params=pltpu.CompilerParams(use_tc_tiling_on_sc=False),
        out_type=jax.ShapeDtypeStruct((BL, _EMB), jnp.float32),
        scratch_types=[
            pltpu.VMEM((rows_w,), jnp.int32),
            pltpu.VMEM((L, _EMB), jnp.float32),
            pltpu.VMEM((_NB, _CHUNK, _EMB), jnp.float32),
        ]
        + [pltpu.SemaphoreType.DMA] * (2 * _NB),
    )
    def k(x_hbm, emb_hbm, pos_hbm, out_hbm, idx_v, pos_v, buf_v, *sems):
        gs = sems[:_NB]
        ws = sems[_NB:]
        cid = lax.axis_index("c")
        sid = lax.axis_index("s")
        wid = sid * NC + cid
        base = wid * rows_w
        pltpu.sync_copy(pos_hbm.at[pl.ds(0, L)], pos_v)
        pltpu.sync_copy(x_hbm.at[pl.ds(base, rows_w)], idx_v)

        def gather_cp(c, slot):
            idx_view = idx_v.at[pl.ds(c * _CHUNK, _CHUNK)]
            return pltpu.make_async_copy(
                emb_hbm.at[idx_view], buf_v.at[slot], gs[slot])

        def wb_cp(c, slot):
            rb = base + c * _CHUNK
            return pltpu.make_async_copy(
                buf_v.at[slot], out_hbm.at[pl.ds(rb, _CHUNK)], ws[slot])

        for b in range(_LOOK):
            gather_cp(b, b).start()

        def outer(o, carry):
            for b in range(_NB):
                c = o * _NB + b
                gather_cp(c, b).wait()
                start = lax.rem(c * _CHUNK, L)
                bufb = buf_v.at[b]

                def row_body(j, _):
                    p = lax.rem(start + j, L)
                    for e in range(_EMB // 16):
                        sl = pl.ds(e * 16, 16)
                        plsc.addupdate(bufb.at[j, sl], pos_v[p, sl])
                    return 0

                lax.fori_loop(0, _CHUNK, row_body, 0, unroll=2)
                wb_cp(c, b).start()
                b2 = (b + _LOOK) % _NB

                @pl.when(c >= _NB - _LOOK)
                def _drain():
                    wb_cp(c - (_NB - _LOOK), b2).wait()

                @pl.when(c + _LOOK < n_chunks)
                def _fire():
                    gather_cp(c + _LOOK, b2).start()
            return carry

        lax.fori_loop(0, n_chunks // _NB, outer, 0)
        for c in range(n_chunks - (_NB - _LOOK), n_chunks):
            wb_cp(c, c % _NB).wait()

    return k


def kernel(x, emb_table, pos_table):
    B, L = x.shape
    BL = B * L
    xf = x.reshape(BL).astype(jnp.int32)
    out = _build(BL, L)(xf, emb_table, pos_table)
    return out.reshape(B, L, _EMB)
